# baseline (speedup 1.0000x reference)
import functools
import math

import jax
import jax.numpy as jnp
from jax import lax
from jax.experimental import pallas as pl
from jax.experimental.pallas import tpu as pltpu
from jax.experimental.pallas import tpu_sc as plsc

N = 2048
D = 768
G = 4
EPG = 2
E = G * EPG
H = 3072
M = 256
NT = N * 2 // M + E
S = NT * M
NC, NS = 2, 16
NW = NC * NS
CHUNK = N // NW
D2 = D // 2
GS = 1.0 / math.sqrt(G)
GROUP_TOP_P = 0.9


def _routing_body(x_ref, wc_ref, bc_ref, w_ref, dest_ref, teid_ref, x32_ref):
    lt = lax.dot_general(wc_ref[...], x_ref[...], (((0,), (1,)), ((), ())),
                         preferred_element_type=jnp.float32)
    lt = lt + bc_ref[...]

    gl = lt[0:4, :]
    gm = jnp.max(gl, axis=0, keepdims=True)
    ge = jnp.exp(gl - gm)
    gp = ge / jnp.sum(ge, axis=0, keepdims=True)
    p = [gp[i:i + 1, :] for i in range(4)]
    mx1 = jnp.maximum(jnp.maximum(p[0], p[1]), jnp.maximum(p[2], p[3]))
    i0 = jnp.where(p[0] >= mx1, 0,
                   jnp.where(p[1] >= mx1, 1, jnp.where(p[2] >= mx1, 2, 3)))
    q = [jnp.where(i0 == j, -1.0, p[j]) for j in range(4)]
    mx2 = jnp.maximum(jnp.maximum(q[0], q[1]), jnp.maximum(q[2], q[3]))
    i1 = jnp.where(q[0] >= mx2, 0,
                   jnp.where(q[1] >= mx2, 1, jnp.where(q[2] >= mx2, 2, 3)))
    gam1 = (mx1 + mx2) <= GROUP_TOP_P
    den = mx1 + jnp.where(gam1, mx2, 0.0) + 1e-9
    grn0 = mx1 / den
    grn1 = jnp.where(gam1, mx2, 0.0) / den

    rp, esel = [], []
    for g in range(G):
        a = lt[4 + 2 * g:5 + 2 * g, :]
        b = lt[5 + 2 * g:6 + 2 * g, :]
        m = jnp.maximum(a, b)
        ea, eb = jnp.exp(a - m), jnp.exp(b - m)
        ssum = ea + eb
        pa, pb = ea / ssum, eb / ssum
        esel.append((pb > pa).astype(jnp.int32))
        pmx = jnp.maximum(pa, pb)
        rp.append(pmx / (pmx + 1e-9))

    def pick(idx, vals):
        r = vals[3]
        for g in (2, 1, 0):
            r = jnp.where(idx == g, vals[g], r)
        return r

    w0 = grn0 * pick(i0, rp) * GS
    w1 = grn1 * pick(i1, rp) * GS
    eid0 = i0 * 2 + pick(i0, esel)
    eid1 = i1 * 2 + pick(i1, esel)

    ioe = lax.broadcasted_iota(jnp.int32, (E, N), 0)
    oh0 = (ioe == eid0).astype(jnp.float32)
    oh1 = (ioe == eid1).astype(jnp.float32) * gam1.astype(jnp.float32)
    tri = (lax.broadcasted_iota(jnp.int32, (N, N), 0)
           < lax.broadcasted_iota(jnp.int32, (N, N), 1)).astype(jnp.float32)
    cum0 = lax.dot_general(oh0, tri, (((1,), (0,)), ((), ())),
                           preferred_element_type=jnp.float32)
    r0 = jnp.sum(cum0 * oh0, axis=0, keepdims=True)
    tot0 = jnp.sum(oh0, axis=1, keepdims=True)
    cum1 = lax.dot_general(oh1, tri, (((1,), (0,)), ((), ())),
                           preferred_element_type=jnp.float32) + tot0
    r1 = jnp.sum(cum1 * oh1, axis=0, keepdims=True)
    tot = tot0 + jnp.sum(oh1, axis=1, keepdims=True)
    pc = jnp.ceil(tot * (1.0 / M)) * M
    triE = (lax.broadcasted_iota(jnp.int32, (E, E), 0)
            > lax.broadcasted_iota(jnp.int32, (E, E), 1)).astype(jnp.float32)
    pstart = lax.dot_general(triE, pc, (((1,), (0,)), ((), ())),
                             preferred_element_type=jnp.float32)
    dest0 = jnp.sum(oh0 * pstart, axis=0, keepdims=True) + r0
    dest1c = jnp.sum(oh1 * pstart, axis=0, keepdims=True) + r1
    dest1 = jnp.where(gam1, dest1c, dest0)

    pend = pstart + pc
    itile = lax.broadcasted_iota(jnp.int32, (E, NT), 1).astype(jnp.float32) * M
    teid = jnp.minimum(
        jnp.sum((pend <= itile).astype(jnp.float32), axis=0, keepdims=True),
        float(E - 1))
    nslots = jnp.sum(pc, axis=0, keepdims=True)
    tvalid = (itile[0:1, :] < nslots).astype(jnp.float32)

    w_ref[0:1, :] = w0
    w_ref[1:2, :] = w1
    dest_ref[0:1, :] = dest0.astype(jnp.int32)
    dest_ref[1:2, :] = dest1.astype(jnp.int32)
    teid_ref[0:1, 0:NT] = teid.astype(jnp.int32)
    teid_ref[0:1, NT:2 * NT] = tvalid.astype(jnp.int32)
    x32_ref[...] = _pack_rows(x_ref[...])


def _route(xf, Wc, bc):
    return pl.pallas_call(
        _routing_body,
        out_shape=(
            jax.ShapeDtypeStruct((2, N), jnp.float32),
            jax.ShapeDtypeStruct((2, N), jnp.int32),
            jax.ShapeDtypeStruct((1, 2 * NT), jnp.int32),
            jax.ShapeDtypeStruct((N, D2), jnp.int32),
        ),
    )(xf, Wc, bc)


@functools.cache
def _build_sc_scatter():
    @functools.partial(
        pl.kernel,
        out_type=jax.ShapeDtypeStruct((S, D2), jnp.int32),
        mesh=plsc.VectorSubcoreMesh(core_axis_name="c", subcore_axis_name="s",
                                    num_cores=NC, num_subcores=NS),
        scratch_types=[
            pltpu.VMEM((CHUNK, D2), jnp.int32),
            pltpu.VMEM((CHUNK,), jnp.int32),
            pltpu.VMEM((CHUNK,), jnp.int32),
            pltpu.SemaphoreType.DMA,
            pltpu.SemaphoreType.DMA,
            pltpu.SemaphoreType.DMA,
        ],
    )
    def sc_scatter(x_hbm, d0_hbm, d1_hbm, xs_hbm, rows_v, d0_v, d1_v,
                   sem_x, sem_i, sem_s):
        wid = lax.axis_index("s") * NC + lax.axis_index("c")
        base = wid * CHUNK
        cx = pltpu.make_async_copy(x_hbm.at[pl.ds(base, CHUNK)], rows_v, sem_x)
        c0 = pltpu.make_async_copy(d0_hbm.at[pl.ds(base, CHUNK)], d0_v, sem_i)
        c1 = pltpu.make_async_copy(d1_hbm.at[pl.ds(base, CHUNK)], d1_v, sem_i)
        cx.start()
        c0.start()
        c1.start()
        cx.wait()
        c0.wait()
        c1.wait()
        s0 = pltpu.make_async_copy(rows_v, xs_hbm.at[d0_v], sem_s)
        s1 = pltpu.make_async_copy(rows_v, xs_hbm.at[d1_v], sem_s)
        s0.start()
        s1.start()
        s0.wait()
        s1.wait()

    return sc_scatter


def _sc_scatter(xf, d0, d1):
    return _build_sc_scatter()(xf, d0, d1)


HH = H // 2


def _pack_rows(y):
    yu = pltpu.bitcast(y.astype(jnp.bfloat16), jnp.uint16)
    lo = yu[:, :D2].astype(jnp.uint32)
    hi = yu[:, D2:].astype(jnp.uint32)
    return pltpu.bitcast(lo | (hi << 16), jnp.int32)


def _unpack_rows(y32):
    yu = pltpu.bitcast(y32, jnp.uint32)
    lo = pltpu.bitcast((yu & 0xFFFF).astype(jnp.uint16), jnp.bfloat16)
    hi = pltpu.bitcast((yu >> 16).astype(jnp.uint16), jnp.bfloat16)
    return jnp.concatenate([lo, hi], axis=1)


def _mm_body(teid_sm, xs_ref, w1a_ref, w1b_ref, w2a_ref, w2b_ref,
             w3a_ref, w3b_ref, b1_ref, b2_ref, b3_ref, ys_ref):
    i = pl.program_id(0)

    @pl.when(teid_sm[NT + i] == 1)
    def _():
        def dot(a, b):
            return lax.dot_general(a, b[0].astype(jnp.bfloat16),
                                   (((1,), (0,)), ((), ())),
                                   preferred_element_type=jnp.float32)

        xb = _unpack_rows(xs_ref[...])
        h1 = (jnp.concatenate([dot(xb, w1a_ref), dot(xb, w1b_ref)], axis=1)
              + b1_ref[0])
        h2 = (jnp.concatenate([dot(xb, w2a_ref), dot(xb, w2b_ref)], axis=1)
              + b2_ref[0])
        h = h1 * jax.nn.sigmoid(h1) * h2
        hb = h.astype(jnp.bfloat16)
        y = dot(hb[:, :HH], w3a_ref) + dot(hb[:, HH:], w3b_ref) + b3_ref[0]
        ys_ref[...] = _pack_rows(y)


def _grouped_mm(teid, xs, W1b, W2b, W3b, b1r, b2r, b3r):
    wspec = [
        pl.BlockSpec((1, D, HH), lambda i, s: (s[i], 0, 0)),
        pl.BlockSpec((1, D, HH), lambda i, s: (s[i], 0, 1)),
    ]
    grid_spec = pltpu.PrefetchScalarGridSpec(
        num_scalar_prefetch=1,
        grid=(NT,),
        in_specs=[pl.BlockSpec((M, D2), lambda i, s: (i, 0))] + wspec + wspec
        + [
            pl.BlockSpec((1, HH, D), lambda i, s: (s[i], 0, 0)),
            pl.BlockSpec((1, HH, D), lambda i, s: (s[i], 1, 0)),
            pl.BlockSpec((1, 1, H), lambda i, s: (s[i], 0, 0)),
            pl.BlockSpec((1, 1, H), lambda i, s: (s[i], 0, 0)),
            pl.BlockSpec((1, 1, D), lambda i, s: (s[i], 0, 0)),
        ],
        out_specs=pl.BlockSpec((M, D2), lambda i, s: (i, 0)),
    )
    return pl.pallas_call(
        _mm_body,
        grid_spec=grid_spec,
        out_shape=jax.ShapeDtypeStruct((S, D2), jnp.int32),
    )(teid, xs, W1b, W1b, W2b, W2b, W3b, W3b, b1r, b2r, b3r)


@functools.cache
def _build_sc_gather():
    @functools.partial(
        pl.kernel,
        out_type=tuple(
            jax.ShapeDtypeStruct((N, D2), jnp.int32) for _ in range(2)),
        mesh=plsc.VectorSubcoreMesh(core_axis_name="c", subcore_axis_name="s",
                                    num_cores=NC, num_subcores=NS),
        scratch_types=[
            pltpu.VMEM((CHUNK, D2), jnp.int32),
            pltpu.VMEM((CHUNK, D2), jnp.int32),
            pltpu.VMEM((CHUNK,), jnp.int32),
            pltpu.VMEM((CHUNK,), jnp.int32),
            pltpu.SemaphoreType.DMA,
            pltpu.SemaphoreType.DMA,
            pltpu.SemaphoreType.DMA,
        ],
    )
    def sc_gather(ys_hbm, d0_hbm, d1_hbm, y0_hbm, y1_hbm,
                  rows_a, rows_b, d0_v, d1_v, sem_i, sem_g, sem_w):
        wid = lax.axis_index("s") * NC + lax.axis_index("c")
        base = wid * CHUNK
        sl = pl.ds(base, CHUNK)
        c0 = pltpu.make_async_copy(d0_hbm.at[sl], d0_v, sem_i)
        c1 = pltpu.make_async_copy(d1_hbm.at[sl], d1_v, sem_i)
        c0.start(); c1.start()
        c0.wait(); c1.wait()
        g0 = pltpu.make_async_copy(ys_hbm.at[d0_v], rows_a, sem_g)
        g0.start()
        g1 = pltpu.make_async_copy(ys_hbm.at[d1_v], rows_b, sem_g)
        g1.start()
        g0.wait()
        w0 = pltpu.make_async_copy(rows_a, y0_hbm.at[sl], sem_w)
        w0.start()
        g1.wait()
        w1 = pltpu.make_async_copy(rows_b, y1_hbm.at[sl], sem_w)
        w1.start()
        w0.wait()
        w1.wait()

    return sc_gather


def _sc_gather(ysflat, d0, d1):
    return _build_sc_gather()(ysflat, d0, d1)


def _combine_body(wc_ref, y0_ref, y1_ref, out_ref):
    wc = wc_ref[...]

    def up(ref):
        return _unpack_rows(ref[...]).astype(jnp.float32)

    out_ref[...] = wc[:, 0:1] * up(y0_ref) + wc[:, 1:2] * up(y1_ref)


def _combine(wcol, ysg0, ysg1):
    TB = 256
    return pl.pallas_call(
        _combine_body,
        grid=(N // TB,),
        in_specs=[pl.BlockSpec((TB, 2), lambda i: (i, 0))] + [
            pl.BlockSpec((TB, D2), lambda i: (i, 0)) for _ in range(2)
        ],
        out_specs=pl.BlockSpec((TB, D), lambda i: (i, 0)),
        out_shape=jax.ShapeDtypeStruct((N, D), jnp.float32),
    )(wcol, ysg0, ysg1)


def kernel(x, Wr, br, Wg, bg, W1, b1, W2, b2, W3, b3):
    Bz, Tz, Dz = x.shape
    xf = x.reshape(-1, Dz)

    Wc = jnp.concatenate([Wr] + [Wg[g] for g in range(G)], axis=1)
    bc = jnp.concatenate([br, bg.reshape(-1)]).reshape(12, 1)
    wpair, dests, teid2, x32 = _route(xf, Wc, bc)
    d0, d1 = dests[0], dests[1]
    teid = teid2[0]
    wcol = wpair.T

    xs = _sc_scatter(x32, d0, d1)

    W1b = W1.reshape(E, D, H)
    W2b = W2.reshape(E, D, H)
    W3b = W3.reshape(E, H, D)
    b1r = b1.reshape(E, 1, H)
    b2r = b2.reshape(E, 1, H)
    b3r = b3.reshape(E, 1, D)
    ys = _grouped_mm(teid, xs, W1b, W2b, W3b, b1r, b2r, b3r)

    g0, g1 = _sc_gather(ys, d0, d1)
    out = _combine(wcol, g0, g1)
    return out.reshape(Bz, Tz, Dz)


# speedup vs baseline: 1.0387x; 1.0387x over previous
import functools
import math

import jax
import jax.numpy as jnp
from jax import lax
from jax.experimental import pallas as pl
from jax.experimental.pallas import tpu as pltpu
from jax.experimental.pallas import tpu_sc as plsc

N = 2048
D = 768
G = 4
EPG = 2
E = G * EPG
H = 3072
M = 256
NT = N * 2 // M + E
S = NT * M
NC, NS = 2, 16
NW = NC * NS
CHUNK = N // NW
D2 = D // 2
GS = 1.0 / math.sqrt(G)
GROUP_TOP_P = 0.9


def _routing_body(x_ref, wc_ref, bc_ref, w_ref, dest_ref, teid_ref, x32_ref):
    lt = lax.dot_general(wc_ref[...], x_ref[...], (((0,), (1,)), ((), ())),
                         preferred_element_type=jnp.float32)
    lt = lt + bc_ref[...]

    gl = lt[0:4, :]
    gm = jnp.max(gl, axis=0, keepdims=True)
    ge = jnp.exp(gl - gm)
    gp = ge / jnp.sum(ge, axis=0, keepdims=True)
    p = [gp[i:i + 1, :] for i in range(4)]
    mx1 = jnp.maximum(jnp.maximum(p[0], p[1]), jnp.maximum(p[2], p[3]))
    i0 = jnp.where(p[0] >= mx1, 0,
                   jnp.where(p[1] >= mx1, 1, jnp.where(p[2] >= mx1, 2, 3)))
    q = [jnp.where(i0 == j, -1.0, p[j]) for j in range(4)]
    mx2 = jnp.maximum(jnp.maximum(q[0], q[1]), jnp.maximum(q[2], q[3]))
    i1 = jnp.where(q[0] >= mx2, 0,
                   jnp.where(q[1] >= mx2, 1, jnp.where(q[2] >= mx2, 2, 3)))
    gam1 = (mx1 + mx2) <= GROUP_TOP_P
    den = mx1 + jnp.where(gam1, mx2, 0.0) + 1e-9
    grn0 = mx1 / den
    grn1 = jnp.where(gam1, mx2, 0.0) / den

    rp, esel = [], []
    for g in range(G):
        a = lt[4 + 2 * g:5 + 2 * g, :]
        b = lt[5 + 2 * g:6 + 2 * g, :]
        m = jnp.maximum(a, b)
        ea, eb = jnp.exp(a - m), jnp.exp(b - m)
        ssum = ea + eb
        pa, pb = ea / ssum, eb / ssum
        esel.append((pb > pa).astype(jnp.int32))
        pmx = jnp.maximum(pa, pb)
        rp.append(pmx / (pmx + 1e-9))

    def pick(idx, vals):
        r = vals[3]
        for g in (2, 1, 0):
            r = jnp.where(idx == g, vals[g], r)
        return r

    w0 = grn0 * pick(i0, rp) * GS
    w1 = grn1 * pick(i1, rp) * GS
    eid0 = i0 * 2 + pick(i0, esel)
    eid1 = i1 * 2 + pick(i1, esel)

    ioe = lax.broadcasted_iota(jnp.int32, (E, N), 0)
    oh0 = (ioe == eid0).astype(jnp.float32)
    oh1 = (ioe == eid1).astype(jnp.float32) * gam1.astype(jnp.float32)
    tri = (lax.broadcasted_iota(jnp.int32, (N, N), 0)
           < lax.broadcasted_iota(jnp.int32, (N, N), 1)).astype(jnp.float32)
    cum0 = lax.dot_general(oh0, tri, (((1,), (0,)), ((), ())),
                           preferred_element_type=jnp.float32)
    r0 = jnp.sum(cum0 * oh0, axis=0, keepdims=True)
    tot0 = jnp.sum(oh0, axis=1, keepdims=True)
    cum1 = lax.dot_general(oh1, tri, (((1,), (0,)), ((), ())),
                           preferred_element_type=jnp.float32) + tot0
    r1 = jnp.sum(cum1 * oh1, axis=0, keepdims=True)
    tot = tot0 + jnp.sum(oh1, axis=1, keepdims=True)
    pc = jnp.ceil(tot * (1.0 / M)) * M
    triE = (lax.broadcasted_iota(jnp.int32, (E, E), 0)
            > lax.broadcasted_iota(jnp.int32, (E, E), 1)).astype(jnp.float32)
    pstart = lax.dot_general(triE, pc, (((1,), (0,)), ((), ())),
                             preferred_element_type=jnp.float32)
    dest0 = jnp.sum(oh0 * pstart, axis=0, keepdims=True) + r0
    dest1c = jnp.sum(oh1 * pstart, axis=0, keepdims=True) + r1
    dest1 = jnp.where(gam1, dest1c, dest0)

    pend = pstart + pc
    itile = lax.broadcasted_iota(jnp.int32, (E, NT), 1).astype(jnp.float32) * M
    teid = jnp.minimum(
        jnp.sum((pend <= itile).astype(jnp.float32), axis=0, keepdims=True),
        float(E - 1))
    nslots = jnp.sum(pc, axis=0, keepdims=True)
    tvalid = (itile[0:1, :] < nslots).astype(jnp.float32)

    w_ref[0:1, :] = w0
    w_ref[1:2, :] = w1
    dest_ref[0:1, :] = dest0.astype(jnp.int32)
    dest_ref[1:2, :] = dest1.astype(jnp.int32)
    teid_ref[0:1, 0:NT] = teid.astype(jnp.int32)
    teid_ref[0:1, NT:2 * NT] = tvalid.astype(jnp.int32)
    x32_ref[...] = _pack_rows(x_ref[...])


def _route(xf, Wc, bc):
    return pl.pallas_call(
        _routing_body,
        out_shape=(
            jax.ShapeDtypeStruct((2, N), jnp.float32),
            jax.ShapeDtypeStruct((2, N), jnp.int32),
            jax.ShapeDtypeStruct((1, 2 * NT), jnp.int32),
            jax.ShapeDtypeStruct((N, D2), jnp.int32),
        ),
    )(xf, Wc, bc)


@functools.cache
def _build_sc_scatter():
    @functools.partial(
        pl.kernel,
        out_type=jax.ShapeDtypeStruct((S, D2), jnp.int32),
        mesh=plsc.VectorSubcoreMesh(core_axis_name="c", subcore_axis_name="s",
                                    num_cores=NC, num_subcores=NS),
        scratch_types=[
            pltpu.VMEM((CHUNK, D2), jnp.int32),
            pltpu.VMEM((CHUNK,), jnp.int32),
            pltpu.VMEM((CHUNK,), jnp.int32),
            pltpu.SemaphoreType.DMA,
            pltpu.SemaphoreType.DMA,
            pltpu.SemaphoreType.DMA,
        ],
    )
    def sc_scatter(x_hbm, d0_hbm, d1_hbm, xs_hbm, rows_v, d0_v, d1_v,
                   sem_x, sem_i, sem_s):
        wid = lax.axis_index("s") * NC + lax.axis_index("c")
        base = wid * CHUNK
        cx = pltpu.make_async_copy(x_hbm.at[pl.ds(base, CHUNK)], rows_v, sem_x)
        c0 = pltpu.make_async_copy(d0_hbm.at[pl.ds(base, CHUNK)], d0_v, sem_i)
        c1 = pltpu.make_async_copy(d1_hbm.at[pl.ds(base, CHUNK)], d1_v, sem_i)
        cx.start()
        c0.start()
        c1.start()
        cx.wait()
        c0.wait()
        c1.wait()
        s0 = pltpu.make_async_copy(rows_v, xs_hbm.at[d0_v], sem_s)
        s1 = pltpu.make_async_copy(rows_v, xs_hbm.at[d1_v], sem_s)
        s0.start()
        s1.start()
        s0.wait()
        s1.wait()

    return sc_scatter


def _sc_scatter(xf, d0, d1):
    return _build_sc_scatter()(xf, d0, d1)


HH = H // 2


def _pack_rows(y):
    yu = pltpu.bitcast(y.astype(jnp.bfloat16), jnp.uint16)
    lo = yu[:, :D2].astype(jnp.uint32)
    hi = yu[:, D2:].astype(jnp.uint32)
    return pltpu.bitcast(lo | (hi << 16), jnp.int32)


def _unpack_rows(y32):
    yu = pltpu.bitcast(y32, jnp.uint32)
    lo = pltpu.bitcast((yu & 0xFFFF).astype(jnp.uint16), jnp.bfloat16)
    hi = pltpu.bitcast((yu >> 16).astype(jnp.uint16), jnp.bfloat16)
    return jnp.concatenate([lo, hi], axis=1)


def _mm_body(teid_sm, xs_ref, w1_ref, w2_ref, w3_ref, b1_ref, b2_ref, b3_ref,
             ys_ref):
    i = pl.program_id(0)

    @pl.when(teid_sm[NT + i] == 1)
    def _():
        xb = _unpack_rows(xs_ref[...])
        h1 = lax.dot_general(xb, w1_ref[0].astype(jnp.bfloat16),
                             (((1,), (0,)), ((), ())),
                             preferred_element_type=jnp.float32) + b1_ref[0]
        h2 = lax.dot_general(xb, w2_ref[0].astype(jnp.bfloat16),
                             (((1,), (0,)), ((), ())),
                             preferred_element_type=jnp.float32) + b2_ref[0]
        h = h1 * jax.nn.sigmoid(h1) * h2
        hb = h.astype(jnp.bfloat16)
        y = lax.dot_general(hb, w3_ref[0].astype(jnp.bfloat16),
                            (((1,), (0,)), ((), ())),
                            preferred_element_type=jnp.float32)
        y = y + b3_ref[0]
        ys_ref[...] = _pack_rows(y)


def _grouped_mm(teid, xs, W1b, W2b, W3b, b1r, b2r, b3r):
    grid_spec = pltpu.PrefetchScalarGridSpec(
        num_scalar_prefetch=1,
        grid=(NT,),
        in_specs=[
            pl.BlockSpec((M, D2), lambda i, s: (i, 0)),
            pl.BlockSpec((1, D, H), lambda i, s: (s[i], 0, 0)),
            pl.BlockSpec((1, D, H), lambda i, s: (s[i], 0, 0)),
            pl.BlockSpec((1, H, D), lambda i, s: (s[i], 0, 0)),
            pl.BlockSpec((1, 1, H), lambda i, s: (s[i], 0, 0)),
            pl.BlockSpec((1, 1, H), lambda i, s: (s[i], 0, 0)),
            pl.BlockSpec((1, 1, D), lambda i, s: (s[i], 0, 0)),
        ],
        out_specs=pl.BlockSpec((M, D2), lambda i, s: (i, 0)),
    )
    return pl.pallas_call(
        _mm_body,
        grid_spec=grid_spec,
        out_shape=jax.ShapeDtypeStruct((S, D2), jnp.int32),
    )(teid, xs, W1b, W2b, W3b, b1r, b2r, b3r)


@functools.cache
def _build_sc_gather():
    @functools.partial(
        pl.kernel,
        out_type=tuple(
            jax.ShapeDtypeStruct((N, D2), jnp.int32) for _ in range(2)),
        mesh=plsc.VectorSubcoreMesh(core_axis_name="c", subcore_axis_name="s",
                                    num_cores=NC, num_subcores=NS),
        scratch_types=[
            pltpu.VMEM((CHUNK, D2), jnp.int32),
            pltpu.VMEM((CHUNK, D2), jnp.int32),
            pltpu.VMEM((CHUNK,), jnp.int32),
            pltpu.VMEM((CHUNK,), jnp.int32),
            pltpu.SemaphoreType.DMA,
            pltpu.SemaphoreType.DMA,
            pltpu.SemaphoreType.DMA,
        ],
    )
    def sc_gather(ys_hbm, d0_hbm, d1_hbm, y0_hbm, y1_hbm,
                  rows_a, rows_b, d0_v, d1_v, sem_i, sem_g, sem_w):
        wid = lax.axis_index("s") * NC + lax.axis_index("c")
        base = wid * CHUNK
        sl = pl.ds(base, CHUNK)
        c0 = pltpu.make_async_copy(d0_hbm.at[sl], d0_v, sem_i)
        c1 = pltpu.make_async_copy(d1_hbm.at[sl], d1_v, sem_i)
        c0.start(); c1.start()
        c0.wait(); c1.wait()
        g0 = pltpu.make_async_copy(ys_hbm.at[d0_v], rows_a, sem_g)
        g0.start()
        g1 = pltpu.make_async_copy(ys_hbm.at[d1_v], rows_b, sem_g)
        g1.start()
        g0.wait()
        w0 = pltpu.make_async_copy(rows_a, y0_hbm.at[sl], sem_w)
        w0.start()
        g1.wait()
        w1 = pltpu.make_async_copy(rows_b, y1_hbm.at[sl], sem_w)
        w1.start()
        w0.wait()
        w1.wait()

    return sc_gather


def _sc_gather(ysflat, d0, d1):
    return _build_sc_gather()(ysflat, d0, d1)


def _combine_body(wc_ref, y0_ref, y1_ref, out_ref):
    wc = wc_ref[...]

    def up(ref):
        return _unpack_rows(ref[...]).astype(jnp.float32)

    out_ref[...] = wc[:, 0:1] * up(y0_ref) + wc[:, 1:2] * up(y1_ref)


def _combine(wcol, ysg0, ysg1):
    TB = 512
    return pl.pallas_call(
        _combine_body,
        grid=(N // TB,),
        in_specs=[pl.BlockSpec((TB, 2), lambda i: (i, 0))] + [
            pl.BlockSpec((TB, D2), lambda i: (i, 0)) for _ in range(2)
        ],
        out_specs=pl.BlockSpec((TB, D), lambda i: (i, 0)),
        out_shape=jax.ShapeDtypeStruct((N, D), jnp.float32),
    )(wcol, ysg0, ysg1)


def kernel(x, Wr, br, Wg, bg, W1, b1, W2, b2, W3, b3):
    Bz, Tz, Dz = x.shape
    xf = x.reshape(-1, Dz)

    Wc = jnp.concatenate([Wr] + [Wg[g] for g in range(G)], axis=1)
    bc = jnp.concatenate([br, bg.reshape(-1)]).reshape(12, 1)
    wpair, dests, teid2, x32 = _route(xf, Wc, bc)
    d0, d1 = dests[0], dests[1]
    teid = teid2[0]
    wcol = wpair.T

    xs = _sc_scatter(x32, d0, d1)

    W1b = W1.reshape(E, D, H)
    W2b = W2.reshape(E, D, H)
    W3b = W3.reshape(E, H, D)
    b1r = b1.reshape(E, 1, H)
    b2r = b2.reshape(E, 1, H)
    b3r = b3.reshape(E, 1, D)
    ys = _grouped_mm(teid, xs, W1b, W2b, W3b, b1r, b2r, b3r)

    g0, g1 = _sc_gather(ys, d0, d1)
    out = _combine(wcol, g0, g1)
    return out.reshape(Bz, Tz, Dz)
